# baseline (speedup 1.0000x reference)
import functools

import jax
import jax.numpy as jnp
from jax import lax
from jax.experimental import pallas as pl
from jax.experimental.pallas import tpu as pltpu
from jax.experimental.pallas import tpu_sc as plsc

VOCAB = 1_000_000
DIM = 64
N_IDX = 16_384

_NC = 2
_NS = 16
_NW = _NC * _NS
_BPW = N_IDX // _NW
_CHUNK = 128
_NCH = _BPW // _CHUNK


def _gather_body(table_hbm, idx_hbm, out_hbm, idx_v, rows_v, sem):
    wid = lax.axis_index("s") * _NC + lax.axis_index("c")
    base = wid * _BPW
    pltpu.sync_copy(idx_hbm.at[pl.ds(wid * _NCH, _NCH)], idx_v)
    copies = [
        pltpu.async_copy(
            table_hbm.at[idx_v.at[j]],
            rows_v.at[pl.ds(j * _CHUNK, _CHUNK)],
            sem,
        )
        for j in range(_NCH)
    ]
    for c in copies:
        c.wait()
    pltpu.sync_copy(rows_v, out_hbm.at[pl.ds(base, _BPW)])


_gather_call = functools.partial(
    pl.kernel,
    mesh=plsc.VectorSubcoreMesh(core_axis_name="c", subcore_axis_name="s"),
    out_type=jax.ShapeDtypeStruct((N_IDX, DIM), jnp.float32),
    scratch_types=[
        pltpu.VMEM((_NCH, _CHUNK), jnp.int32),
        pltpu.VMEM((_BPW, DIM), jnp.float32),
        pltpu.SemaphoreType.DMA,
    ],
    compiler_params=pltpu.CompilerParams(use_tc_tiling_on_sc=False),
)(_gather_body)


def kernel(x, indices):
    idx2d = indices.reshape(_NW * _NCH, _CHUNK)
    return _gather_call(x, idx2d)


# speedup vs baseline: 1.0326x; 1.0326x over previous
import functools

import jax
import jax.numpy as jnp
from jax import lax
from jax.experimental import pallas as pl
from jax.experimental.pallas import tpu as pltpu
from jax.experimental.pallas import tpu_sc as plsc

VOCAB = 1_000_000
DIM = 64
N_IDX = 16_384

_NC = 2
_NS = 16
_NW = _NC * _NS
_BPW = N_IDX // _NW
_K = 16
_NB = _BPW // _K


def _gather_body(x_hbm, idx_hbm, out_hbm, idx_v, sem):
    wid = lax.axis_index("s") * _NC + lax.axis_index("c")
    base = wid * _BPW
    pltpu.sync_copy(idx_hbm.at[pl.ds(base, _BPW)], idx_v)

    def batch(b, carry):
        i0 = b * _K
        idx_vec = idx_v[pl.ds(i0, _K)]
        copies = []
        for j in range(_K):
            row = idx_vec[j]
            copies.append(
                pltpu.async_copy(
                    x_hbm.at[pl.ds(row, 1)],
                    out_hbm.at[pl.ds(base + i0 + j, 1)],
                    sem,
                )
            )
        for c in copies:
            c.wait()
        return carry

    lax.fori_loop(0, _NB, batch, 0)


_gather_call = functools.partial(
    pl.kernel,
    mesh=plsc.VectorSubcoreMesh(core_axis_name="c", subcore_axis_name="s"),
    out_type=jax.ShapeDtypeStruct((N_IDX, DIM), jnp.float32),
    scratch_types=[
        pltpu.VMEM((_BPW,), jnp.int32),
        pltpu.SemaphoreType.DMA,
    ],
)(_gather_body)


def kernel(x, indices):
    return _gather_call(x, indices)
